# Initial kernel scaffold; baseline (speedup 1.0000x reference)
import jax
import jax.numpy as jnp
from jax.experimental import pallas as pl


def kernel(indices, table):
    raise NotImplementedError("write your pallas kernel here")


# speedup vs baseline: 1.8320x; 1.8320x over previous
import functools

import jax
import jax.numpy as jnp
from jax import lax
from jax.experimental import pallas as pl
from jax.experimental.pallas import tpu as pltpu
from jax.experimental.pallas import tpu_sc as plsc

VOCAB_ = 1000000
EMBED_ = 64
NUM_CORES = 2
NUM_SUBCORES = 16
NW = NUM_CORES * NUM_SUBCORES

CHUNK = 512


def _emb_kernel(n_total, table_hbm, idx_hbm, out_hbm, idx_v, rows_v, sem_g):
    b_per_w = n_total // NW
    n_chunks = b_per_w // CHUNK
    wid = lax.axis_index("s") * NUM_CORES + lax.axis_index("c")
    base = pl.multiple_of(wid * b_per_w, 8)

    pltpu.sync_copy(idx_hbm.at[pl.ds(base, b_per_w)], idx_v)

    def body(c, _):
        off = pl.multiple_of(c * CHUNK, 8)
        idx_chunk = idx_v.at[pl.ds(off, CHUNK)]
        pltpu.async_copy(table_hbm.at[idx_chunk], rows_v, sem_g).wait()
        pltpu.sync_copy(rows_v, out_hbm.at[pl.ds(base + off, CHUNK)])
        return ()

    lax.fori_loop(0, n_chunks, body, (), unroll=False)


def kernel(indices, table):
    batch, hist = indices.shape
    n_total = batch * hist
    idx_flat = indices.reshape(n_total).astype(jnp.int32)
    b_per_w = n_total // NW

    mesh = plsc.VectorSubcoreMesh(
        core_axis_name="c", subcore_axis_name="s",
        num_cores=NUM_CORES, num_subcores=NUM_SUBCORES,
    )
    k = pl.kernel(
        functools.partial(_emb_kernel, n_total),
        out_type=jax.ShapeDtypeStruct((n_total, EMBED_), jnp.float32),
        mesh=mesh,
        scratch_types=[
            pltpu.VMEM((b_per_w,), jnp.int32),
            pltpu.VMEM((CHUNK, EMBED_), jnp.float32),
            pltpu.SemaphoreType.DMA,
        ],
        compiler_params=pltpu.CompilerParams(use_tc_tiling_on_sc=False),
    )
    out = k(table, idx_flat)
    return out.reshape(batch, hist, EMBED_)


# speedup vs baseline: 1.8778x; 1.0250x over previous
import functools

import jax
import jax.numpy as jnp
from jax import lax
from jax.experimental import pallas as pl
from jax.experimental.pallas import tpu as pltpu
from jax.experimental.pallas import tpu_sc as plsc

EMBED_ = 64
NUM_CORES = 2
NUM_SUBCORES = 16
NW = NUM_CORES * NUM_SUBCORES

CHUNK = 512


def _emb_kernel(n_total, table_hbm, idx_hbm, out_hbm,
                idx_v, rows0, rows1, sg0, sg1, ss0, ss1):
    b_per_w = n_total // NW
    n_chunks = b_per_w // CHUNK
    n_pairs = n_chunks // 2
    wid = lax.axis_index("s") * NUM_CORES + lax.axis_index("c")
    base = pl.multiple_of(wid * b_per_w, 8)

    pltpu.sync_copy(idx_hbm.at[pl.ds(base, b_per_w)], idx_v)

    def g_copy(c, rows, sem):
        off = pl.multiple_of(c * CHUNK, 8)
        return pltpu.make_async_copy(
            table_hbm.at[idx_v.at[pl.ds(off, CHUNK)]], rows, sem)

    def s_copy(c, rows, sem):
        off = pl.multiple_of(c * CHUNK, 8)
        return pltpu.make_async_copy(
            rows, out_hbm.at[pl.ds(base + off, CHUNK)], sem)

    g_copy(0, rows0, sg0).start()

    def pair(p, _):
        c0 = p * 2
        c1 = c0 + 1
        @pl.when(p > 0)
        def _():
            s_copy(c1, rows1, ss1).wait()

        g_copy(c1, rows1, sg1).start()
        g_copy(c0, rows0, sg0).wait()
        s_copy(c0, rows0, ss0).start()
        s_copy(c0, rows0, ss0).wait()

        @pl.when(p + 1 < n_pairs)
        def _():
            g_copy(c0 + 2, rows0, sg0).start()

        g_copy(c1, rows1, sg1).wait()
        s_copy(c1, rows1, ss1).start()
        return ()

    lax.fori_loop(0, n_pairs, pair, (), unroll=False)
    s_copy(n_chunks - 1, rows1, ss1).wait()


def kernel(indices, table):
    batch, hist = indices.shape
    n_total = batch * hist
    idx_flat = indices.reshape(n_total).astype(jnp.int32)
    b_per_w = n_total // NW

    mesh = plsc.VectorSubcoreMesh(
        core_axis_name="c", subcore_axis_name="s",
        num_cores=NUM_CORES, num_subcores=NUM_SUBCORES,
    )
    k = pl.kernel(
        functools.partial(_emb_kernel, n_total),
        out_type=jax.ShapeDtypeStruct((n_total, EMBED_), jnp.float32),
        mesh=mesh,
        scratch_types=[
            pltpu.VMEM((b_per_w,), jnp.int32),
            pltpu.VMEM((CHUNK, EMBED_), jnp.float32),
            pltpu.VMEM((CHUNK, EMBED_), jnp.float32),
            pltpu.SemaphoreType.DMA,
            pltpu.SemaphoreType.DMA,
            pltpu.SemaphoreType.DMA,
            pltpu.SemaphoreType.DMA,
        ],
        compiler_params=pltpu.CompilerParams(use_tc_tiling_on_sc=False),
    )
    out = k(table, idx_flat)
    return out.reshape(batch, hist, EMBED_)
